# baseline (speedup 1.0000x reference)
import jax
import jax.numpy as jnp
from jax import lax
from jax.experimental import pallas as pl
from jax.experimental.pallas import tpu as pltpu
from jax.experimental.pallas import tpu_sc as plsc

_LANES = 16
_NW = 32
_CH = 4000


def _reproj_body(cidx_hbm, pidx_hbm, kct_hbm, px_hbm, py_hbm, pz_hbm,
                 projx_hbm, projy_hbm, kct_v, cidx_v, pidx_v, px_v, py_v,
                 pz_v, resx_v, resy_v, sem_lin, sem_pa, sem_pb, sem_oa,
                 sem_ob):
    n_obs = cidx_hbm.shape[0]
    n_cams = kct_hbm.shape[0] // 10
    nchunks = n_obs // _CH
    cpt = (nchunks + _NW - 1) // _NW
    wid = lax.axis_index("s") * 2 + lax.axis_index("c")

    pltpu.sync_copy(kct_hbm, kct_v)

    def lin_copies(chunk, slot):
        src = pl.ds(chunk * _CH, _CH)
        dst = pl.ds(slot * _CH, _CH)
        return (
            pltpu.make_async_copy(cidx_hbm.at[src], cidx_v.at[dst], sem_lin),
            pltpu.make_async_copy(pidx_hbm.at[src], pidx_v.at[dst], sem_lin),
        )

    def load_linear(chunk, slot):
        for c in lin_copies(chunk, slot):
            c.start()
        for c in lin_copies(chunk, slot):
            c.wait()

    def p_copies(slot, sem):
        blk = pl.ds(slot * _CH, _CH)
        idx = pidx_v.at[blk]
        return (
            pltpu.make_async_copy(px_hbm.at[idx], px_v.at[blk], sem),
            pltpu.make_async_copy(py_hbm.at[idx], py_v.at[blk], sem),
            pltpu.make_async_copy(pz_hbm.at[idx], pz_v.at[blk], sem),
        )

    def out_copies(chunk, slot, sem):
        blk = pl.ds(slot * _CH, _CH)
        out_sl = pl.ds(chunk * _CH, _CH)
        return (
            pltpu.make_async_copy(resx_v.at[blk], projx_hbm.at[out_sl], sem),
            pltpu.make_async_copy(resy_v.at[blk], projy_hbm.at[out_sl], sem),
        )

    def compute(chunk, slot):
        soff = slot * _CH

        @pl.loop(0, _CH // _LANES, unroll=4)
        def _compute(i):
            sl = pl.ds(soff + i * _LANES, _LANES)
            cam = cidx_v[sl]

            def kc(j):
                return plsc.load_gather(
                    kct_v.at[pl.ds(j * n_cams, n_cams)], [cam])

            f = kc(0)
            k1 = kc(1)
            k2 = kc(2)
            tx = kc(3)
            ty = kc(4)
            tz = kc(5)
            qx = kc(6)
            qy = kc(7)
            qz = kc(8)
            qw = kc(9)

            px = px_v[sl]
            py = py_v[sl]
            pz = pz_v[sl]

            t1x = qy * pz - qz * py + qw * px
            t1y = qz * px - qx * pz + qw * py
            t1z = qx * py - qy * px + qw * pz
            cx = px + 2.0 * (qy * t1z - qz * t1y) + tx
            cy = py + 2.0 * (qz * t1x - qx * t1z) + ty
            cz = pz + 2.0 * (qx * t1y - qy * t1x) + tz

            inv = -1.0 / cz
            nx = cx * inv
            ny = cy * inv
            r = nx * nx + ny * ny
            fd = f * (1.0 + k1 * r + k2 * (r * r))

            resx_v[sl] = fd * nx
            resy_v[sl] = fd * ny

    def stage(t, u, slot, sem_this, sem_next, sem_out_this):
        chunk = wid + t * _NW
        nxt = chunk + _NW

        @pl.when(u > 0)
        def _():
            for c in out_copies(chunk - 2 * _NW, slot, sem_out_this):
                c.wait()

        @pl.when(nxt < nchunks)
        def _():
            load_linear(nxt, 1 - slot)
            for c in p_copies(1 - slot, sem_next):
                c.start()

        @pl.when(chunk < nchunks)
        def _():
            for c in p_copies(slot, sem_this):
                c.wait()
            compute(chunk, slot)
            for c in out_copies(chunk, slot, sem_out_this):
                c.start()

    load_linear(wid, 0)
    for c in p_copies(0, sem_pa):
        c.start()

    @pl.loop(0, cpt // 2)
    def _pair(u):
        stage(2 * u, u, 0, sem_pa, sem_pb, sem_oa)
        stage(2 * u + 1, u, 1, sem_pb, sem_pa, sem_ob)

    @pl.when(wid + (cpt - 2) * _NW < nchunks)
    def _():
        for c in out_copies(wid + (cpt - 2) * _NW, 0, sem_oa):
            c.wait()

    @pl.when(wid + (cpt - 1) * _NW < nchunks)
    def _():
        for c in out_copies(wid + (cpt - 1) * _NW, 1, sem_ob):
            c.wait()


def kernel(observe, cidx, pidx, K, C, P):
    n_obs = observe.shape[0]
    cidx = cidx.astype(jnp.int32)
    pidx = pidx.astype(jnp.int32)
    KCt = jnp.concatenate([K, C], axis=1).T.reshape(-1)
    px, py, pz = P[:, 0], P[:, 1], P[:, 2]

    nchunks = n_obs // _CH
    assert nchunks * _CH == n_obs and ((nchunks + _NW - 1) // _NW) % 2 == 0

    mesh = plsc.VectorSubcoreMesh(core_axis_name="c", subcore_axis_name="s")
    kfun = pl.kernel(
        _reproj_body,
        out_type=(
            jax.ShapeDtypeStruct((n_obs,), jnp.float32),
            jax.ShapeDtypeStruct((n_obs,), jnp.float32),
        ),
        mesh=mesh,
        compiler_params=pltpu.CompilerParams(needs_layout_passes=False),
        scratch_types=[
            pltpu.VMEM((KCt.shape[0],), jnp.float32),
            pltpu.VMEM((2 * _CH,), jnp.int32),
            pltpu.VMEM((2 * _CH,), jnp.int32),
            pltpu.VMEM((2 * _CH,), jnp.float32),
            pltpu.VMEM((2 * _CH,), jnp.float32),
            pltpu.VMEM((2 * _CH,), jnp.float32),
            pltpu.VMEM((2 * _CH,), jnp.float32),
            pltpu.VMEM((2 * _CH,), jnp.float32),
            pltpu.SemaphoreType.DMA,
            pltpu.SemaphoreType.DMA,
            pltpu.SemaphoreType.DMA,
            pltpu.SemaphoreType.DMA,
            pltpu.SemaphoreType.DMA,
        ],
    )
    projx, projy = kfun(cidx, pidx, KCt, px, py, pz)
    return jnp.stack([projx, projy], axis=-1) - observe
